# baseline (speedup 1.0000x reference)
import jax
import jax.numpy as jnp
from jax import lax
from jax.experimental import pallas as pl

N_NODES = 10000
DEG = 32
D = 128
BN = 80


def _fused_body(x_ref, neigh_ref, wx_ref, bx_ref, wn_ref, bn_ref, out_ref):
    nb = neigh_ref[...].reshape(BN, DEG, D)
    agg = jnp.sum(nb, axis=1) * (1.0 / DEG)
    h_x = lax.dot_general(
        x_ref[...], wx_ref[...], (((1,), (1,)), ((), ())),
        preferred_element_type=jnp.float32)
    h_n = lax.dot_general(
        agg, wn_ref[...], (((1,), (1,)), ((), ())),
        preferred_element_type=jnp.float32)
    out_ref[:, :D] = jnp.maximum(h_x + bx_ref[...], 0.0)
    out_ref[:, D:] = jnp.maximum(h_n + bn_ref[...], 0.0)


@jax.jit
def _fused(x, neigh, W_x, b_x, W_n, b_n):
    return pl.pallas_call(
        _fused_body,
        grid=(N_NODES // BN,),
        in_specs=[
            pl.BlockSpec((BN, D), lambda i: (i, 0)),
            pl.BlockSpec((BN * DEG, D), lambda i: (i, 0)),
            pl.BlockSpec((D, D), lambda i: (0, 0)),
            pl.BlockSpec((1, D), lambda i: (0, 0)),
            pl.BlockSpec((D, D), lambda i: (0, 0)),
            pl.BlockSpec((1, D), lambda i: (0, 0)),
        ],
        out_specs=pl.BlockSpec((BN, 2 * D), lambda i: (i, 0)),
        out_shape=jax.ShapeDtypeStruct((N_NODES, 2 * D), jnp.float32),
    )(x, neigh, W_x, b_x, W_n, b_n)


def kernel(x, neigh, W_x, b_x, W_n, b_n):
    return _fused(x, neigh, W_x.reshape(D, D), b_x.reshape(1, D),
                  W_n.reshape(D, D), b_n.reshape(1, D))


# speedup vs baseline: 1.9853x; 1.9853x over previous
import jax
import jax.numpy as jnp
from jax import lax
from jax.experimental import pallas as pl

N_NODES = 10000
DEG = 32
D = 128
BN = 400


def _fused_body(x_ref, neigh_ref, wx_ref, bx_ref, wn_ref, bn_ref, out_ref):
    nb = neigh_ref[...].reshape(BN, DEG, D)
    agg = jnp.sum(nb, axis=1) * (1.0 / DEG)
    h_x = lax.dot_general(
        x_ref[...], wx_ref[...], (((1,), (1,)), ((), ())),
        preferred_element_type=jnp.float32)
    h_n = lax.dot_general(
        agg, wn_ref[...], (((1,), (1,)), ((), ())),
        preferred_element_type=jnp.float32)
    out_ref[:, :D] = jnp.maximum(h_x + bx_ref[...], 0.0)
    out_ref[:, D:] = jnp.maximum(h_n + bn_ref[...], 0.0)


@jax.jit
def _fused(x, neigh, W_x, b_x, W_n, b_n):
    return pl.pallas_call(
        _fused_body,
        grid=(N_NODES // BN,),
        in_specs=[
            pl.BlockSpec((BN, D), lambda i: (i, 0)),
            pl.BlockSpec((BN * DEG, D), lambda i: (i, 0)),
            pl.BlockSpec((D, D), lambda i: (0, 0)),
            pl.BlockSpec((1, D), lambda i: (0, 0)),
            pl.BlockSpec((D, D), lambda i: (0, 0)),
            pl.BlockSpec((1, D), lambda i: (0, 0)),
        ],
        out_specs=pl.BlockSpec((BN, 2 * D), lambda i: (i, 0)),
        out_shape=jax.ShapeDtypeStruct((N_NODES, 2 * D), jnp.float32),
    )(x, neigh, W_x, b_x, W_n, b_n)


def kernel(x, neigh, W_x, b_x, W_n, b_n):
    return _fused(x, neigh, W_x.reshape(D, D), b_x.reshape(1, D),
                  W_n.reshape(D, D), b_n.reshape(1, D))
